# Initial kernel scaffold; baseline (speedup 1.0000x reference)
import jax
import jax.numpy as jnp
from jax.experimental import pallas as pl


def kernel(x, coords, W, b, gamma, beta):
    raise NotImplementedError("write your pallas kernel here")


# speedup vs baseline: 4.4544x; 4.4544x over previous
import functools

import jax
import jax.numpy as jnp
from jax import lax
from jax.experimental import pallas as pl
from jax.experimental.pallas import tpu as pltpu
from jax.experimental.pallas import tpu_sc as plsc


def _sc_gather(x2, c16, gidx, C):
    info = plsc.get_sparse_core_info()
    NW = info.num_cores * info.num_subcores
    R = gidx.shape[0]
    rpw = R // NW
    G = 32
    n_chunks = rpw // G
    mesh = plsc.VectorSubcoreMesh(core_axis_name="c", subcore_axis_name="s")

    @functools.partial(
        pl.kernel,
        mesh=mesh,
        out_type=(
            jax.ShapeDtypeStruct((R, C), jnp.float32),
            jax.ShapeDtypeStruct((R, 128), jnp.float32),
        ),
        scratch_types=[
            pltpu.VMEM((G,), jnp.int32),
            pltpu.VMEM((G, C), jnp.float32),
            pltpu.VMEM((G, 128), jnp.float32),
            pltpu.SemaphoreType.DMA,
            pltpu.SemaphoreType.DMA,
        ],
    )
    def k(x_hbm, c_hbm, idx_hbm, xs_hbm, cs_hbm, idx_v, rows_v, crows_v,
          sx, sc):
        wid = lax.axis_index("s") * info.num_cores + lax.axis_index("c")
        base = wid * rpw

        def body(ci, _):
            off = base + ci * G
            pltpu.sync_copy(idx_hbm.at[pl.ds(off, G)], idx_v)
            cx = pltpu.async_copy(x_hbm.at[idx_v], rows_v, sx)
            cc = pltpu.async_copy(c_hbm.at[idx_v], crows_v, sc)
            cx.wait()
            cc.wait()
            pltpu.sync_copy(rows_v, xs_hbm.at[pl.ds(off, G)])
            pltpu.sync_copy(crows_v, cs_hbm.at[pl.ds(off, G)])
            return 0

        lax.fori_loop(0, n_chunks, body, 0)

    return k(x2, c16, gidx)


def _tc_pool_proj(xs, cs, W, bias, gamma, beta):
    M = xs.shape[0] // 2
    C = xs.shape[1]
    D = W.shape[1]
    BM = 256
    nb = M // BM

    def body(e_ref, o_ref, ce_ref, co_ref, w_ref, b_ref, g_ref, be_ref,
             h_ref, cp_ref):
        xm = jnp.maximum(e_ref[...], o_ref[...])
        h = jnp.dot(xm, w_ref[...],
                    preferred_element_type=jnp.float32,
                    precision=lax.Precision.HIGHEST)
        h = h + b_ref[...]
        mean = jnp.mean(h, axis=1, keepdims=True)
        var = jnp.mean((h - mean) ** 2, axis=1, keepdims=True)
        hn = (h - mean) * lax.rsqrt(var + 1e-6)
        h_ref[...] = hn * g_ref[...] + be_ref[...]
        cp_ref[...] = (ce_ref[...] + co_ref[...]) * 0.5

    return pl.pallas_call(
        body,
        grid=(nb,),
        in_specs=[
            pl.BlockSpec((BM, C), lambda i: (i, 0)),
            pl.BlockSpec((BM, C), lambda i: (i + nb, 0)),
            pl.BlockSpec((BM, 128), lambda i: (i, 0)),
            pl.BlockSpec((BM, 128), lambda i: (i + nb, 0)),
            pl.BlockSpec((C, D), lambda i: (0, 0)),
            pl.BlockSpec((1, D), lambda i: (0, 0)),
            pl.BlockSpec((1, D), lambda i: (0, 0)),
            pl.BlockSpec((1, D), lambda i: (0, 0)),
        ],
        out_specs=[
            pl.BlockSpec((BM, D), lambda i: (i, 0)),
            pl.BlockSpec((BM, 128), lambda i: (i, 0)),
        ],
        out_shape=[
            jax.ShapeDtypeStruct((M, D), jnp.float32),
            jax.ShapeDtypeStruct((M, 128), jnp.float32),
        ],
    )(xs, xs, cs, cs, W, bias, gamma, beta)


def kernel(x, coords, W, b, gamma, beta):
    B, N, C = x.shape
    D = W.shape[1]
    N_out = N // 2
    M = B * N_out

    eta = coords[..., 0]
    sort_idx = jnp.argsort(eta, axis=1).astype(jnp.int32)
    gi = sort_idx + (jnp.arange(B, dtype=jnp.int32) * N)[:, None]
    gidx = jnp.concatenate(
        [gi[:, 0::2].reshape(-1), gi[:, 1::2].reshape(-1)])

    x2 = x.reshape(B * N, C)
    c16 = jnp.concatenate(
        [coords.reshape(B * N, 2),
         jnp.zeros((B * N, 126), dtype=jnp.float32)], axis=1)

    xs, cs = _sc_gather(x2, c16, gidx, C)

    h, cp = _tc_pool_proj(xs, cs, W, b.reshape(1, D),
                          gamma.reshape(1, D), beta.reshape(1, D))
    return (h.reshape(B, N_out, D), cp[:, :2].reshape(B, N_out, 2))
